# baseline (speedup 1.0000x reference)
import jax
import jax.numpy as jnp
from jax import lax
from jax.experimental import pallas as pl
from jax.experimental.pallas import tpu as pltpu
from jax.experimental.pallas import tpu_sc as plsc

N = 10000
E = 160000
D = 256
DH = 128
NC = 2
NS = 16
NW = NC * NS
L = 16

CK1 = 80
NCH1 = E // CK1
NCK1 = 63
E1 = NW * NCK1 * CK1

CK3 = 128
NCK3 = 80
E3 = NS * NCK3 * CK3

_NEG = -1e30


def _seg_max_rmw(arr, d16, v16):
    def cond(carry):
        return jnp.any(carry[0])

    def body(carry):
        act, v = carry
        cur = plsc.load_gather(arr, [d16])
        new = jnp.maximum(cur, v)
        plsc.store_scatter(arr, [d16], new, mask=act)
        back = plsc.load_gather(arr, [d16])
        return act & (back < new), v

    lax.while_loop(cond, body, (jnp.full((L,), True), v16))


def _k1_body(x_hbm, e1_hbm, scores_hbm, smaxp_hbm,
             ebuf, xs, xd, sbuf, tb, smaxp, sems, semd):
    c = lax.axis_index("c")
    s = lax.axis_index("s")
    w = s * NC + c
    rows = jnp.arange(L, dtype=jnp.int32)

    def init_i(i, _):
        smaxp[pl.ds(i * L, L)] = jnp.full((L,), _NEG, jnp.float32)
        return 0
    lax.fori_loop(0, N // L, init_i, 0)

    def issue(ci, b):
        cid = w * NCK1 + ci
        pltpu.sync_copy(e1_hbm.at[cid], ebuf.at[b])
        pltpu.async_copy(x_hbm.at[ebuf.at[b].at[0]], xs.at[b], sems.at[b])
        pltpu.async_copy(x_hbm.at[ebuf.at[b].at[1]], xd.at[b], semd.at[b])

    def compute(ci, b):
        cid = w * NCK1 + ci
        pltpu.make_async_copy(x_hbm.at[ebuf.at[b].at[0]], xs.at[b],
                              sems.at[b]).wait()
        pltpu.make_async_copy(x_hbm.at[ebuf.at[b].at[1]], xd.at[b],
                              semd.at[b]).wait()

        @pl.when(cid < NCH1)
        def _():
            for g in range(CK1 // L):
                def edge(ee, _):
                    e = g * L + ee
                    accs = []
                    for j in range(D // L):
                        p = (xs[b, e, pl.ds(j * L, L)] *
                             xd[b, e, pl.ds(j * L, L)])
                        if j < 4:
                            accs.append(p)
                        else:
                            accs[j % 4] = accs[j % 4] + p
                    tb[pl.ds(ee * L, L)] = ((accs[0] + accs[1]) +
                                            (accs[2] + accs[3]))
                    return 0
                lax.fori_loop(0, L, edge, 0, unroll=4)
                s16 = jnp.zeros((L,), jnp.float32)
                for j in range(L):
                    s16 = s16 + plsc.load_gather(tb, [rows * L + j])
                sbuf[pl.ds(g * L, L)] = s16
                d16 = ebuf[b, 1, pl.ds(g * L, L)]
                _seg_max_rmw(smaxp, d16, s16)
            pltpu.sync_copy(sbuf, scores_hbm.at[pl.ds(cid * CK1, CK1)])

    issue(0, 0)

    def pair(i, _):
        issue(2 * i + 1, 1)
        compute(2 * i, 0)
        issue(2 * i + 2, 0)
        compute(2 * i + 1, 1)
        return 0
    lax.fori_loop(0, (NCK1 - 1) // 2, pair, 0)
    compute(NCK1 - 1, 0)

    pltpu.sync_copy(smaxp, smaxp_hbm.at[w])


def _k3_body(ft2_hbm, e3_hbm, smax_hbm,
             agg2_hbm, den_hbm,
             ebuf, exv, ftv, smaxp, zbuf, dzero,
             agg_sh, den_sh, sems, scsem, dsem):
    c = lax.axis_index("c")
    s = lax.axis_index("s")

    def zinit(i, _):
        for j in range(DH // L):
            zbuf[i, pl.ds(j * L, L)] = jnp.zeros((L,), jnp.float32)
        return 0
    lax.fori_loop(0, 25, zinit, 0)

    def zcopy(r, _):
        pltpu.sync_copy(zbuf, agg_sh.at[pl.ds(s * 625 + r * 25, 25)])
        return 0
    lax.fori_loop(0, 25, zcopy, 0)

    @pl.when(c == 0)
    def _():
        def dz(i, _):
            dzero[pl.ds(i * L, L)] = jnp.zeros((L,), jnp.float32)
            return 0
        lax.fori_loop(0, 62, dz, 0)
        dzero[pl.ds(984, L)] = jnp.zeros((L,), jnp.float32)

        @pl.when(s < 10)
        def _():
            pltpu.sync_copy(dzero, den_sh.at[pl.ds(s * 1000, 1000)])

    pltpu.sync_copy(smax_hbm, smaxp)
    plsc.subcore_barrier()

    def agg_desc(b):
        return pltpu.make_async_copy(ftv.at[b], agg_sh.at[ebuf.at[b].at[1]],
                                     scsem.at[b])

    def den_desc(b):
        return pltpu.make_async_copy(exv.at[b], den_sh.at[ebuf.at[b].at[1]],
                                     dsem.at[b])

    def issue(ci, b):
        @pl.when(ci >= 2)
        def _():
            agg_desc(b).wait()

            @pl.when(c == 0)
            def _():
                den_desc(b).wait()

        cid = s * NCK3 + ci
        pltpu.sync_copy(e3_hbm.at[cid], ebuf.at[b])
        pltpu.async_copy(ft2_hbm.at[c].at[ebuf.at[b].at[0]], ftv.at[b],
                         sems.at[b])

    def compute(b):
        pltpu.make_async_copy(ft2_hbm.at[c].at[ebuf.at[b].at[0]], ftv.at[b],
                              sems.at[b]).wait()
        for g in range(CK3 // L):
            s16 = plsc.bitcast(ebuf[b, 2, pl.ds(g * L, L)], jnp.float32)
            d16 = ebuf[b, 1, pl.ds(g * L, L)]
            m16 = plsc.load_gather(smaxp, [d16])
            exv[b, pl.ds(g * L, L)] = jnp.exp(s16 - m16)

        def edge(e, _):
            ex = plsc.load_gather(exv.at[b], [jnp.full((L,), 0, jnp.int32) + e])
            for j in range(DH // L):
                ftv[b, e, pl.ds(j * L, L)] = ftv[b, e, pl.ds(j * L, L)] * ex
            return 0
        lax.fori_loop(0, CK3, edge, 0, unroll=2)

        @pl.when(c == 0)
        def _():
            pltpu.async_copy(exv.at[b], den_sh.at[ebuf.at[b].at[1]],
                             dsem.at[b], add=True)

        pltpu.async_copy(ftv.at[b], agg_sh.at[ebuf.at[b].at[1]],
                         scsem.at[b], add=True)

    issue(0, 0)

    def pair(i, _):
        issue(2 * i + 1, 1)
        compute(0)
        issue(2 * i + 2, 0)
        compute(1)
        return 0
    lax.fori_loop(0, NCK3 // 2 - 1, pair, 0)
    issue(NCK3 - 1, 1)
    compute(0)
    compute(1)

    agg_desc(0).wait()
    agg_desc(1).wait()

    @pl.when(c == 0)
    def _():
        den_desc(0).wait()
        den_desc(1).wait()

    plsc.subcore_barrier()

    def ocopy(r, _):
        sl = pl.ds(s * 625 + r * 125, 125)
        pltpu.sync_copy(agg_sh.at[sl], agg2_hbm.at[c].at[sl])
        return 0
    lax.fori_loop(0, 5, ocopy, 0)

    @pl.when((c == 0) & (s < 10))
    def _():
        sl = pl.ds(s * 1000, 1000)
        pltpu.sync_copy(den_sh.at[sl], den_hbm.at[sl])


def _tc_mm_body(x_ref, W_ref, Wr_ref, br_ref, ft2_ref, res_ref):
    xb = x_ref[...]
    dn = (((1,), (0,)), ((), ()))
    f = lax.dot_general(xb, W_ref[...], dn,
                        precision=lax.Precision.HIGHEST,
                        preferred_element_type=jnp.float32)
    ft2_ref[0] = f[:, :DH]
    ft2_ref[1] = f[:, DH:]
    r = lax.dot_general(xb, Wr_ref[...], dn,
                        precision=lax.Precision.HIGHEST,
                        preferred_element_type=jnp.float32) + br_ref[...]
    res_ref[...] = jnp.maximum(r, 0.0)


def _tc_smax_body(smaxp_ref, smax_ref):
    smax_ref[...] = jnp.max(smaxp_ref[...], axis=0, keepdims=True)


def _tc_final_body(agg2_ref, den_ref, res_ref, g_ref, b_ref, out_ref):
    agg = jnp.concatenate([agg2_ref[0], agg2_ref[1]], axis=1)
    den = den_ref[...]
    safe = den > 0.0
    y = jnp.where(safe, agg / jnp.where(safe, den, 1.0), 0.0) + res_ref[...]
    mean = jnp.mean(y, axis=0, keepdims=True)
    var = jnp.mean((y - mean) ** 2, axis=0, keepdims=True)
    out_ref[...] = (y - mean) / jnp.sqrt(var + 1e-5) * g_ref[...] + b_ref[...]


def kernel(x, edge_index, W, W_res, b_res, gamma, beta):
    src = edge_index[0]
    dst = edge_index[1]

    mesh = plsc.VectorSubcoreMesh(core_axis_name="c", subcore_axis_name="s")
    sc_params = pltpu.CompilerParams(use_tc_tiling_on_sc=False,
                                     needs_layout_passes=False)

    pad1 = E1 - E
    src1 = jnp.concatenate([src, jnp.zeros((pad1,), jnp.int32)])
    dst1 = jnp.concatenate([dst, jnp.zeros((pad1,), jnp.int32)])
    e1 = (jnp.stack([src1, dst1])
          .reshape(2, NW * NCK1, CK1).transpose(1, 0, 2))

    scores, smax_part = pl.kernel(
        _k1_body,
        out_type=(jax.ShapeDtypeStruct((E,), jnp.float32),
                  jax.ShapeDtypeStruct((NW, N), jnp.float32)),
        mesh=mesh,
        compiler_params=sc_params,
        scratch_types=[
            pltpu.VMEM((2, 2, CK1), jnp.int32),
            pltpu.VMEM((2, CK1, D), jnp.float32),
            pltpu.VMEM((2, CK1, D), jnp.float32),
            pltpu.VMEM((CK1,), jnp.float32),
            pltpu.VMEM((L * L,), jnp.float32),
            pltpu.VMEM((N,), jnp.float32),
            pltpu.SemaphoreType.DMA((2,)),
            pltpu.SemaphoreType.DMA((2,)),
        ],
    )(x, e1)

    RB = 1000
    ft2, res = pl.pallas_call(
        _tc_mm_body,
        grid=(N // RB,),
        in_specs=[
            pl.BlockSpec((RB, D), lambda i: (i, 0)),
            pl.BlockSpec((D, D), lambda i: (0, 0)),
            pl.BlockSpec((D, D), lambda i: (0, 0)),
            pl.BlockSpec((1, D), lambda i: (0, 0)),
        ],
        out_specs=[
            pl.BlockSpec((NC, RB, DH), lambda i: (0, i, 0)),
            pl.BlockSpec((RB, D), lambda i: (i, 0)),
        ],
        out_shape=[
            jax.ShapeDtypeStruct((NC, N, DH), jnp.float32),
            jax.ShapeDtypeStruct((N, D), jnp.float32),
        ],
    )(x, W, W_res, b_res[None, :])

    smax2 = pl.pallas_call(
        _tc_smax_body,
        out_shape=jax.ShapeDtypeStruct((1, N), jnp.float32),
    )(smax_part)
    smax = smax2.reshape((N,))

    pad3 = E3 - E
    src3 = jnp.concatenate([src, jnp.zeros((pad3,), jnp.int32)])
    dst3 = jnp.concatenate([dst, jnp.zeros((pad3,), jnp.int32)])
    sc3 = lax.bitcast_convert_type(
        jnp.concatenate([scores, jnp.full((pad3,), -jnp.inf, jnp.float32)]),
        jnp.int32)
    e3 = (jnp.stack([src3, dst3, sc3])
          .reshape(3, NS * NCK3, CK3).transpose(1, 0, 2))

    agg2, den = pl.kernel(
        _k3_body,
        out_type=(jax.ShapeDtypeStruct((NC, N, DH), jnp.float32),
                  jax.ShapeDtypeStruct((N,), jnp.float32)),
        mesh=mesh,
        compiler_params=sc_params,
        scratch_types=[
            pltpu.VMEM((2, 3, CK3), jnp.int32),
            pltpu.VMEM((2, CK3), jnp.float32),
            pltpu.VMEM((2, CK3, DH), jnp.float32),
            pltpu.VMEM((N,), jnp.float32),
            pltpu.VMEM((25, DH), jnp.float32),
            pltpu.VMEM((1000,), jnp.float32),
            pltpu.VMEM_SHARED((N, DH), jnp.float32),
            pltpu.VMEM_SHARED((N,), jnp.float32),
            pltpu.SemaphoreType.DMA((2,)),
            pltpu.SemaphoreType.DMA((2,)),
            pltpu.SemaphoreType.DMA((2,)),
        ],
    )(ft2, e3, smax)

    out = pl.pallas_call(
        _tc_final_body,
        out_shape=jax.ShapeDtypeStruct((N, D), jnp.float32),
    )(agg2, den.reshape((N, 1)), res, gamma[None, :], beta[None, :])
    return out


# speedup vs baseline: 1.0212x; 1.0212x over previous
import jax
import jax.numpy as jnp
from jax import lax
from jax.experimental import pallas as pl
from jax.experimental.pallas import tpu as pltpu
from jax.experimental.pallas import tpu_sc as plsc

N = 10000
E = 160000
D = 256
DH = 128
NC = 2
NS = 16
NW = NC * NS
L = 16

CK1 = 80
NCH1 = E // CK1
NCK1 = 63
E1 = NW * NCK1 * CK1

CK3 = 128
NCK3 = 80
E3 = NS * NCK3 * CK3

_NEG = -1e30


def _seg_max_rmw(arr, d16, v16):
    def cond(carry):
        return jnp.any(carry[0])

    def body(carry):
        act, v = carry
        cur = plsc.load_gather(arr, [d16])
        new = jnp.maximum(cur, v)
        plsc.store_scatter(arr, [d16], new, mask=act)
        back = plsc.load_gather(arr, [d16])
        return act & (back < new), v

    lax.while_loop(cond, body, (jnp.full((L,), True), v16))


def _k1_body(x_hbm, e1_hbm, scores_hbm, smaxp_hbm,
             ebuf, xs, xd, sbuf, tb, smaxp, sems, semd):
    c = lax.axis_index("c")
    s = lax.axis_index("s")
    w = s * NC + c
    rows = jnp.arange(L, dtype=jnp.int32)

    def init_i(i, _):
        smaxp[pl.ds(i * L, L)] = jnp.full((L,), _NEG, jnp.float32)
        return 0
    lax.fori_loop(0, N // L, init_i, 0)

    def issue(ci, b):
        cid = w * NCK1 + ci
        pltpu.sync_copy(e1_hbm.at[cid], ebuf.at[b])
        pltpu.async_copy(x_hbm.at[ebuf.at[b].at[0]], xs.at[b], sems.at[b])
        pltpu.async_copy(x_hbm.at[ebuf.at[b].at[1]], xd.at[b], semd.at[b])

    def compute(ci, b):
        cid = w * NCK1 + ci
        pltpu.make_async_copy(x_hbm.at[ebuf.at[b].at[0]], xs.at[b],
                              sems.at[b]).wait()
        pltpu.make_async_copy(x_hbm.at[ebuf.at[b].at[1]], xd.at[b],
                              semd.at[b]).wait()

        @pl.when(cid < NCH1)
        def _():
            for g in range(CK1 // L):
                def edge(ee, _):
                    e = g * L + ee
                    accs = []
                    for j in range(D // L):
                        p = (xs[b, e, pl.ds(j * L, L)] *
                             xd[b, e, pl.ds(j * L, L)])
                        if j < 4:
                            accs.append(p)
                        else:
                            accs[j % 4] = accs[j % 4] + p
                    tb[pl.ds(ee * L, L)] = ((accs[0] + accs[1]) +
                                            (accs[2] + accs[3]))
                    return 0
                lax.fori_loop(0, L, edge, 0)
                s16 = jnp.zeros((L,), jnp.float32)
                for j in range(L):
                    s16 = s16 + plsc.load_gather(tb, [rows * L + j])
                sbuf[pl.ds(g * L, L)] = s16
                d16 = ebuf[b, 1, pl.ds(g * L, L)]
                _seg_max_rmw(smaxp, d16, s16)
            pltpu.sync_copy(sbuf, scores_hbm.at[pl.ds(cid * CK1, CK1)])

    issue(0, 0)

    def pair(i, _):
        issue(2 * i + 1, 1)
        compute(2 * i, 0)
        issue(2 * i + 2, 0)
        compute(2 * i + 1, 1)
        return 0
    lax.fori_loop(0, (NCK1 - 1) // 2, pair, 0)
    compute(NCK1 - 1, 0)

    pltpu.sync_copy(smaxp, smaxp_hbm.at[w])


def _k3_body(ft2_hbm, e3_hbm, smax_hbm,
             agg2_hbm, den_hbm,
             ebuf, exv, ftv, smaxp, zbuf, dzero,
             agg_sh, den_sh, sems, scsem, dsem):
    c = lax.axis_index("c")
    s = lax.axis_index("s")

    def zinit(i, _):
        for j in range(DH // L):
            zbuf[i, pl.ds(j * L, L)] = jnp.zeros((L,), jnp.float32)
        return 0
    lax.fori_loop(0, 25, zinit, 0)

    def zcopy(r, _):
        pltpu.sync_copy(zbuf, agg_sh.at[pl.ds(s * 625 + r * 25, 25)])
        return 0
    lax.fori_loop(0, 25, zcopy, 0)

    @pl.when(c == 0)
    def _():
        def dz(i, _):
            dzero[pl.ds(i * L, L)] = jnp.zeros((L,), jnp.float32)
            return 0
        lax.fori_loop(0, 62, dz, 0)
        dzero[pl.ds(984, L)] = jnp.zeros((L,), jnp.float32)

        @pl.when(s < 10)
        def _():
            pltpu.sync_copy(dzero, den_sh.at[pl.ds(s * 1000, 1000)])

    pltpu.sync_copy(smax_hbm, smaxp)
    plsc.subcore_barrier()

    def agg_desc(b):
        return pltpu.make_async_copy(ftv.at[b], agg_sh.at[ebuf.at[b].at[1]],
                                     scsem.at[b])

    def den_desc(b):
        return pltpu.make_async_copy(exv.at[b], den_sh.at[ebuf.at[b].at[1]],
                                     dsem.at[b])

    def issue(ci, b):
        @pl.when(ci >= 2)
        def _():
            agg_desc(b).wait()

            @pl.when(c == 0)
            def _():
                den_desc(b).wait()

        cid = s * NCK3 + ci
        pltpu.sync_copy(e3_hbm.at[cid], ebuf.at[b])
        pltpu.async_copy(ft2_hbm.at[c].at[ebuf.at[b].at[0]], ftv.at[b],
                         sems.at[b])

    def compute(b):
        pltpu.make_async_copy(ft2_hbm.at[c].at[ebuf.at[b].at[0]], ftv.at[b],
                              sems.at[b]).wait()
        for g in range(CK3 // L):
            s16 = plsc.bitcast(ebuf[b, 2, pl.ds(g * L, L)], jnp.float32)
            d16 = ebuf[b, 1, pl.ds(g * L, L)]
            m16 = plsc.load_gather(smaxp, [d16])
            exv[b, pl.ds(g * L, L)] = jnp.exp(s16 - m16)

        def edge(e, _):
            ex = plsc.load_gather(exv.at[b], [jnp.full((L,), 0, jnp.int32) + e])
            for j in range(DH // L):
                ftv[b, e, pl.ds(j * L, L)] = ftv[b, e, pl.ds(j * L, L)] * ex
            return 0
        lax.fori_loop(0, CK3, edge, 0, unroll=2)

        @pl.when(c == 0)
        def _():
            pltpu.async_copy(exv.at[b], den_sh.at[ebuf.at[b].at[1]],
                             dsem.at[b], add=True)

        pltpu.async_copy(ftv.at[b], agg_sh.at[ebuf.at[b].at[1]],
                         scsem.at[b], add=True)

    issue(0, 0)

    def pair(i, _):
        issue(2 * i + 1, 1)
        compute(0)
        issue(2 * i + 2, 0)
        compute(1)
        return 0
    lax.fori_loop(0, NCK3 // 2 - 1, pair, 0)
    issue(NCK3 - 1, 1)
    compute(0)
    compute(1)

    agg_desc(0).wait()
    agg_desc(1).wait()

    @pl.when(c == 0)
    def _():
        den_desc(0).wait()
        den_desc(1).wait()

    plsc.subcore_barrier()

    def ocopy(r, _):
        sl = pl.ds(s * 625 + r * 125, 125)
        pltpu.sync_copy(agg_sh.at[sl], agg2_hbm.at[c].at[sl])
        return 0
    lax.fori_loop(0, 5, ocopy, 0)

    @pl.when((c == 0) & (s < 10))
    def _():
        sl = pl.ds(s * 1000, 1000)
        pltpu.sync_copy(den_sh.at[sl], den_hbm.at[sl])


def _tc_mm_body(x_ref, W_ref, Wr_ref, br_ref, ft2_ref, res_ref):
    xb = x_ref[...]
    dn = (((1,), (0,)), ((), ()))
    f = lax.dot_general(xb, W_ref[...], dn,
                        precision=lax.Precision.HIGHEST,
                        preferred_element_type=jnp.float32)
    ft2_ref[0] = f[:, :DH]
    ft2_ref[1] = f[:, DH:]
    r = lax.dot_general(xb, Wr_ref[...], dn,
                        precision=lax.Precision.HIGHEST,
                        preferred_element_type=jnp.float32) + br_ref[...]
    res_ref[...] = jnp.maximum(r, 0.0)


def _tc_smax_body(smaxp_ref, smax_ref):
    smax_ref[...] = jnp.max(smaxp_ref[...], axis=0, keepdims=True)


def _tc_final_body(agg2_ref, den_ref, res_ref, g_ref, b_ref, out_ref):
    agg = jnp.concatenate([agg2_ref[0], agg2_ref[1]], axis=1)
    den = den_ref[...]
    safe = den > 0.0
    y = jnp.where(safe, agg / jnp.where(safe, den, 1.0), 0.0) + res_ref[...]
    mean = jnp.mean(y, axis=0, keepdims=True)
    var = jnp.mean((y - mean) ** 2, axis=0, keepdims=True)
    out_ref[...] = (y - mean) / jnp.sqrt(var + 1e-5) * g_ref[...] + b_ref[...]


def kernel(x, edge_index, W, W_res, b_res, gamma, beta):
    src = edge_index[0]
    dst = edge_index[1]

    mesh = plsc.VectorSubcoreMesh(core_axis_name="c", subcore_axis_name="s")
    sc_params = pltpu.CompilerParams(use_tc_tiling_on_sc=False,
                                     needs_layout_passes=False)

    pad1 = E1 - E
    src1 = jnp.concatenate([src, jnp.zeros((pad1,), jnp.int32)])
    dst1 = jnp.concatenate([dst, jnp.zeros((pad1,), jnp.int32)])
    e1 = (jnp.stack([src1, dst1])
          .reshape(2, NW * NCK1, CK1).transpose(1, 0, 2))

    scores, smax_part = pl.kernel(
        _k1_body,
        out_type=(jax.ShapeDtypeStruct((E,), jnp.float32),
                  jax.ShapeDtypeStruct((NW, N), jnp.float32)),
        mesh=mesh,
        compiler_params=sc_params,
        scratch_types=[
            pltpu.VMEM((2, 2, CK1), jnp.int32),
            pltpu.VMEM((2, CK1, D), jnp.float32),
            pltpu.VMEM((2, CK1, D), jnp.float32),
            pltpu.VMEM((CK1,), jnp.float32),
            pltpu.VMEM((L * L,), jnp.float32),
            pltpu.VMEM((N,), jnp.float32),
            pltpu.SemaphoreType.DMA((2,)),
            pltpu.SemaphoreType.DMA((2,)),
        ],
    )(x, e1)

    RB = 1000
    ft2, res = pl.pallas_call(
        _tc_mm_body,
        grid=(N // RB,),
        in_specs=[
            pl.BlockSpec((RB, D), lambda i: (i, 0)),
            pl.BlockSpec((D, D), lambda i: (0, 0)),
            pl.BlockSpec((D, D), lambda i: (0, 0)),
            pl.BlockSpec((1, D), lambda i: (0, 0)),
        ],
        out_specs=[
            pl.BlockSpec((NC, RB, DH), lambda i: (0, i, 0)),
            pl.BlockSpec((RB, D), lambda i: (i, 0)),
        ],
        out_shape=[
            jax.ShapeDtypeStruct((NC, N, DH), jnp.float32),
            jax.ShapeDtypeStruct((N, D), jnp.float32),
        ],
    )(x, W, W_res, b_res[None, :])

    smax2 = pl.pallas_call(
        _tc_smax_body,
        out_shape=jax.ShapeDtypeStruct((1, N), jnp.float32),
    )(smax_part)
    smax = smax2.reshape((N,))

    pad3 = E3 - E
    src3 = jnp.concatenate([src, jnp.zeros((pad3,), jnp.int32)])
    dst3 = jnp.concatenate([dst, jnp.zeros((pad3,), jnp.int32)])
    sc3 = lax.bitcast_convert_type(
        jnp.concatenate([scores, jnp.full((pad3,), -jnp.inf, jnp.float32)]),
        jnp.int32)
    e3 = (jnp.stack([src3, dst3, sc3])
          .reshape(3, NS * NCK3, CK3).transpose(1, 0, 2))

    agg2, den = pl.kernel(
        _k3_body,
        out_type=(jax.ShapeDtypeStruct((NC, N, DH), jnp.float32),
                  jax.ShapeDtypeStruct((N,), jnp.float32)),
        mesh=mesh,
        compiler_params=sc_params,
        scratch_types=[
            pltpu.VMEM((2, 3, CK3), jnp.int32),
            pltpu.VMEM((2, CK3), jnp.float32),
            pltpu.VMEM((2, CK3, DH), jnp.float32),
            pltpu.VMEM((N,), jnp.float32),
            pltpu.VMEM((25, DH), jnp.float32),
            pltpu.VMEM((1000,), jnp.float32),
            pltpu.VMEM_SHARED((N, DH), jnp.float32),
            pltpu.VMEM_SHARED((N,), jnp.float32),
            pltpu.SemaphoreType.DMA((2,)),
            pltpu.SemaphoreType.DMA((2,)),
            pltpu.SemaphoreType.DMA((2,)),
        ],
    )(ft2, e3, smax)

    out = pl.pallas_call(
        _tc_final_body,
        out_shape=jax.ShapeDtypeStruct((N, D), jnp.float32),
    )(agg2, den.reshape((N, 1)), res, gamma[None, :], beta[None, :])
    return out


# speedup vs baseline: 1.1046x; 1.0817x over previous
import jax
import jax.numpy as jnp
from jax import lax
from jax.experimental import pallas as pl
from jax.experimental.pallas import tpu as pltpu
from jax.experimental.pallas import tpu_sc as plsc

N = 10000
E = 160000
D = 256
DH = 128
NC = 2
NS = 16
NW = NC * NS
L = 16

CK1 = 80
NCH1 = E // CK1
NCK1 = 63
E1 = NW * NCK1 * CK1

CK3 = 128
NCK3 = 80
E3 = NS * NCK3 * CK3

_NEG = -1e30


def _seg_max_rmw(arr, d16, v16):
    def cond(carry):
        return jnp.any(carry[0])

    def body(carry):
        act, v = carry
        cur = plsc.load_gather(arr, [d16])
        new = jnp.maximum(cur, v)
        plsc.store_scatter(arr, [d16], new, mask=act)
        back = plsc.load_gather(arr, [d16])
        return act & (back < new), v

    lax.while_loop(cond, body, (jnp.full((L,), True), v16))


def _k1_body(x_hbm, src_hbm, dst_hbm, scores_hbm, smaxp_hbm,
             srcv, dstv, xs, xd, sbuf, tb, smaxp, sems, semd):
    c = lax.axis_index("c")
    s = lax.axis_index("s")
    w = s * NC + c
    base = w * (E // NW)
    rows = jnp.arange(L, dtype=jnp.int32)

    def init_i(i, _):
        smaxp[pl.ds(i * L, L)] = jnp.full((L,), _NEG, jnp.float32)
        return 0
    lax.fori_loop(0, N // L, init_i, 0)

    def off_of(ci):
        return jnp.minimum(base + ci * CK1, E - CK1)

    def issue(ci, b):
        off = off_of(ci)
        pltpu.sync_copy(src_hbm.at[pl.ds(off, CK1)], srcv.at[b])
        pltpu.sync_copy(dst_hbm.at[pl.ds(off, CK1)], dstv.at[b])
        pltpu.async_copy(x_hbm.at[srcv.at[b]], xs.at[b], sems.at[b])
        pltpu.async_copy(x_hbm.at[dstv.at[b]], xd.at[b], semd.at[b])

    def compute(ci, b):
        pltpu.make_async_copy(x_hbm.at[srcv.at[b]], xs.at[b],
                              sems.at[b]).wait()
        pltpu.make_async_copy(x_hbm.at[dstv.at[b]], xd.at[b],
                              semd.at[b]).wait()
        for g in range(CK1 // L):
            def edge(ee, _):
                e = g * L + ee
                accs = []
                for j in range(D // L):
                    p = (xs[b, e, pl.ds(j * L, L)] *
                         xd[b, e, pl.ds(j * L, L)])
                    if j < 4:
                        accs.append(p)
                    else:
                        accs[j % 4] = accs[j % 4] + p
                tb[pl.ds(ee * L, L)] = ((accs[0] + accs[1]) +
                                        (accs[2] + accs[3]))
                return 0
            lax.fori_loop(0, L, edge, 0)
            s16 = jnp.zeros((L,), jnp.float32)
            for j in range(L):
                s16 = s16 + plsc.load_gather(tb, [rows * L + j])
            sbuf[pl.ds(g * L, L)] = s16
            d16 = dstv[b, pl.ds(g * L, L)]
            _seg_max_rmw(smaxp, d16, s16)
        pltpu.sync_copy(sbuf, scores_hbm.at[pl.ds(off_of(ci), CK1)])

    issue(0, 0)

    def pair(i, _):
        issue(2 * i + 1, 1)
        compute(2 * i, 0)
        issue(2 * i + 2, 0)
        compute(2 * i + 1, 1)
        return 0
    lax.fori_loop(0, (NCK1 - 1) // 2, pair, 0)
    compute(NCK1 - 1, 0)

    pltpu.sync_copy(smaxp, smaxp_hbm.at[w])


def _k3_body(ft2_hbm, e3_hbm, smax_hbm,
             agg2_hbm, den_hbm,
             ebuf, exv, ftv, smaxp, zbuf, dzero,
             agg_sh, den_sh, sems, scsem, dsem):
    c = lax.axis_index("c")
    s = lax.axis_index("s")

    def zinit(i, _):
        for j in range(DH // L):
            zbuf[i, pl.ds(j * L, L)] = jnp.zeros((L,), jnp.float32)
        return 0
    lax.fori_loop(0, 25, zinit, 0)

    def zcopy(r, _):
        pltpu.sync_copy(zbuf, agg_sh.at[pl.ds(s * 625 + r * 25, 25)])
        return 0
    lax.fori_loop(0, 25, zcopy, 0)

    @pl.when(c == 0)
    def _():
        def dz(i, _):
            dzero[pl.ds(i * L, L)] = jnp.zeros((L,), jnp.float32)
            return 0
        lax.fori_loop(0, 62, dz, 0)
        dzero[pl.ds(984, L)] = jnp.zeros((L,), jnp.float32)

        @pl.when(s < 10)
        def _():
            pltpu.sync_copy(dzero, den_sh.at[pl.ds(s * 1000, 1000)])

    pltpu.sync_copy(smax_hbm, smaxp)
    plsc.subcore_barrier()

    def agg_desc(b):
        return pltpu.make_async_copy(ftv.at[b], agg_sh.at[ebuf.at[b].at[1]],
                                     scsem.at[b])

    def den_desc(b):
        return pltpu.make_async_copy(exv.at[b], den_sh.at[ebuf.at[b].at[1]],
                                     dsem.at[b])

    def issue(ci, b):
        @pl.when(ci >= 2)
        def _():
            agg_desc(b).wait()

            @pl.when(c == 0)
            def _():
                den_desc(b).wait()

        cid = s * NCK3 + ci
        pltpu.sync_copy(e3_hbm.at[cid], ebuf.at[b])
        pltpu.async_copy(ft2_hbm.at[c].at[ebuf.at[b].at[0]], ftv.at[b],
                         sems.at[b])

    def compute(b):
        pltpu.make_async_copy(ft2_hbm.at[c].at[ebuf.at[b].at[0]], ftv.at[b],
                              sems.at[b]).wait()
        for g in range(CK3 // L):
            s16 = plsc.bitcast(ebuf[b, 2, pl.ds(g * L, L)], jnp.float32)
            d16 = ebuf[b, 1, pl.ds(g * L, L)]
            m16 = plsc.load_gather(smaxp, [d16])
            exv[b, pl.ds(g * L, L)] = jnp.exp(s16 - m16)

        def edge(e, _):
            ex = plsc.load_gather(exv.at[b], [jnp.full((L,), 0, jnp.int32) + e])
            for j in range(DH // L):
                ftv[b, e, pl.ds(j * L, L)] = ftv[b, e, pl.ds(j * L, L)] * ex
            return 0
        lax.fori_loop(0, CK3, edge, 0, unroll=2)

        @pl.when(c == 0)
        def _():
            pltpu.async_copy(exv.at[b], den_sh.at[ebuf.at[b].at[1]],
                             dsem.at[b], add=True)

        pltpu.async_copy(ftv.at[b], agg_sh.at[ebuf.at[b].at[1]],
                         scsem.at[b], add=True)

    issue(0, 0)

    def pair(i, _):
        issue(2 * i + 1, 1)
        compute(0)
        issue(2 * i + 2, 0)
        compute(1)
        return 0
    lax.fori_loop(0, NCK3 // 2 - 1, pair, 0)
    issue(NCK3 - 1, 1)
    compute(0)
    compute(1)

    agg_desc(0).wait()
    agg_desc(1).wait()

    @pl.when(c == 0)
    def _():
        den_desc(0).wait()
        den_desc(1).wait()

    plsc.subcore_barrier()

    def ocopy(r, _):
        sl = pl.ds(s * 625 + r * 125, 125)
        pltpu.sync_copy(agg_sh.at[sl], agg2_hbm.at[c].at[sl])
        return 0
    lax.fori_loop(0, 5, ocopy, 0)

    @pl.when((c == 0) & (s < 10))
    def _():
        sl = pl.ds(s * 1000, 1000)
        pltpu.sync_copy(den_sh.at[sl], den_hbm.at[sl])


def _tc_mm_body(x_ref, W_ref, Wr_ref, br_ref, ft2_ref, res_ref):
    xb = x_ref[...]
    dn = (((1,), (0,)), ((), ()))
    f = lax.dot_general(xb, W_ref[...], dn,
                        precision=lax.Precision.HIGHEST,
                        preferred_element_type=jnp.float32)
    ft2_ref[0] = f[:, :DH]
    ft2_ref[1] = f[:, DH:]
    r = lax.dot_general(xb, Wr_ref[...], dn,
                        precision=lax.Precision.HIGHEST,
                        preferred_element_type=jnp.float32) + br_ref[...]
    res_ref[...] = jnp.maximum(r, 0.0)


def _tc_smax_body(smaxp_ref, smax_ref):
    smax_ref[...] = jnp.max(smaxp_ref[...], axis=0, keepdims=True)


def _tc_final_body(agg2_ref, den_ref, res_ref, g_ref, b_ref, out_ref):
    agg = jnp.concatenate([agg2_ref[0], agg2_ref[1]], axis=1)
    den = den_ref[...]
    safe = den > 0.0
    y = jnp.where(safe, agg / jnp.where(safe, den, 1.0), 0.0) + res_ref[...]
    mean = jnp.mean(y, axis=0, keepdims=True)
    var = jnp.mean((y - mean) ** 2, axis=0, keepdims=True)
    out_ref[...] = (y - mean) / jnp.sqrt(var + 1e-5) * g_ref[...] + b_ref[...]


def kernel(x, edge_index, W, W_res, b_res, gamma, beta):
    src = edge_index[0]
    dst = edge_index[1]

    mesh = plsc.VectorSubcoreMesh(core_axis_name="c", subcore_axis_name="s")
    sc_params = pltpu.CompilerParams(use_tc_tiling_on_sc=False,
                                     needs_layout_passes=False)

    scores, smax_part = pl.kernel(
        _k1_body,
        out_type=(jax.ShapeDtypeStruct((E,), jnp.float32),
                  jax.ShapeDtypeStruct((NW, N), jnp.float32)),
        mesh=mesh,
        compiler_params=sc_params,
        scratch_types=[
            pltpu.VMEM((2, CK1), jnp.int32),
            pltpu.VMEM((2, CK1), jnp.int32),
            pltpu.VMEM((2, CK1, D), jnp.float32),
            pltpu.VMEM((2, CK1, D), jnp.float32),
            pltpu.VMEM((CK1,), jnp.float32),
            pltpu.VMEM((L * L,), jnp.float32),
            pltpu.VMEM((N,), jnp.float32),
            pltpu.SemaphoreType.DMA((2,)),
            pltpu.SemaphoreType.DMA((2,)),
        ],
    )(x, src, dst)

    RB = 1000
    ft2, res = pl.pallas_call(
        _tc_mm_body,
        grid=(N // RB,),
        in_specs=[
            pl.BlockSpec((RB, D), lambda i: (i, 0)),
            pl.BlockSpec((D, D), lambda i: (0, 0)),
            pl.BlockSpec((D, D), lambda i: (0, 0)),
            pl.BlockSpec((1, D), lambda i: (0, 0)),
        ],
        out_specs=[
            pl.BlockSpec((NC, RB, DH), lambda i: (0, i, 0)),
            pl.BlockSpec((RB, D), lambda i: (i, 0)),
        ],
        out_shape=[
            jax.ShapeDtypeStruct((NC, N, DH), jnp.float32),
            jax.ShapeDtypeStruct((N, D), jnp.float32),
        ],
    )(x, W, W_res, b_res[None, :])

    smax2 = pl.pallas_call(
        _tc_smax_body,
        out_shape=jax.ShapeDtypeStruct((1, N), jnp.float32),
    )(smax_part)
    smax = smax2.reshape((N,))

    pad3 = E3 - E
    src3 = jnp.concatenate([src, jnp.zeros((pad3,), jnp.int32)])
    dst3 = jnp.concatenate([dst, jnp.zeros((pad3,), jnp.int32)])
    sc3 = lax.bitcast_convert_type(
        jnp.concatenate([scores, jnp.full((pad3,), -jnp.inf, jnp.float32)]),
        jnp.int32)
    e3 = (jnp.stack([src3, dst3, sc3])
          .reshape(3, NS * NCK3, CK3).transpose(1, 0, 2))

    agg2, den = pl.kernel(
        _k3_body,
        out_type=(jax.ShapeDtypeStruct((NC, N, DH), jnp.float32),
                  jax.ShapeDtypeStruct((N,), jnp.float32)),
        mesh=mesh,
        compiler_params=sc_params,
        scratch_types=[
            pltpu.VMEM((2, 3, CK3), jnp.int32),
            pltpu.VMEM((2, CK3), jnp.float32),
            pltpu.VMEM((2, CK3, DH), jnp.float32),
            pltpu.VMEM((N,), jnp.float32),
            pltpu.VMEM((25, DH), jnp.float32),
            pltpu.VMEM((1000,), jnp.float32),
            pltpu.VMEM_SHARED((N, DH), jnp.float32),
            pltpu.VMEM_SHARED((N,), jnp.float32),
            pltpu.SemaphoreType.DMA((2,)),
            pltpu.SemaphoreType.DMA((2,)),
            pltpu.SemaphoreType.DMA((2,)),
        ],
    )(ft2, e3, smax)

    out = pl.pallas_call(
        _tc_final_body,
        out_shape=jax.ShapeDtypeStruct((N, D), jnp.float32),
    )(agg2, den.reshape((N, 1)), res, gamma[None, :], beta[None, :])
    return out
